# baseline (speedup 1.0000x reference)
import functools

import jax
import jax.numpy as jnp
from jax import lax
from jax.experimental import pallas as pl
from jax.experimental.pallas import tpu as pltpu
from jax.experimental.pallas import tpu_sc as plsc

H = 256
HH = 128
_BN = 1000
_BE = 2000
_CH = 80
_NP = 10240


def _pack_edge_pairs(x):
    xb = jax.lax.bitcast_convert_type(x, jnp.uint32)
    r = (xb + jnp.uint32(0x7FFF) + ((xb >> 16) & jnp.uint32(1))) >> 16
    r3 = r.reshape(r.shape[0] // 2, 2, r.shape[1])
    c = r3[:, 0, :] | (r3[:, 1, :] << 16)
    return jax.lax.bitcast_convert_type(c, jnp.int32)


def _pre_body(x_ref, w_ref, b_ref, s2s_ref, dst_ref):
    x = x_ref[0]
    h = jnp.dot(x, w_ref[0], preferred_element_type=jnp.float32, precision=lax.Precision.DEFAULT) + b_ref[0]
    s2s = jnp.dot(h, w_ref[1], preferred_element_type=jnp.float32, precision=lax.Precision.DEFAULT) + b_ref[1]
    s2s_ref[0] = s2s
    h = jnp.dot(x, w_ref[2], preferred_element_type=jnp.float32, precision=lax.Precision.DEFAULT) + b_ref[2]
    dst = jnp.dot(h, w_ref[3], preferred_element_type=jnp.float32, precision=lax.Precision.DEFAULT) + b_ref[3]
    dst_ref[0, 0] = dst[:, :HH]
    dst_ref[0, 1] = dst[:, HH:]


def _pre(scalar, w4, b4):
    B, N, _ = scalar.shape
    return pl.pallas_call(
        _pre_body,
        grid=(B, N // _BN),
        in_specs=[
            pl.BlockSpec((1, _BN, H), lambda b, i: (b, i, 0)),
            pl.BlockSpec((4, H, H), lambda b, i: (0, 0, 0)),
            pl.BlockSpec((4, 1, H), lambda b, i: (0, 0, 0)),
        ],
        out_specs=[
            pl.BlockSpec((1, _BN, H), lambda b, i: (b, i, 0)),
            pl.BlockSpec((1, 2, _BN, HH), lambda b, i: (b, 0, i, 0)),
        ],
        out_shape=[
            jax.ShapeDtypeStruct((B, N, H), jnp.float32),
            jax.ShapeDtypeStruct((B, 2, N, HH), jnp.float32),
        ],
    )(scalar, w4, b4)


def _efp_body(ef_ref, we_ref, out_ref):
    y = jnp.dot(ef_ref[0], we_ref[...], preferred_element_type=jnp.float32, precision=lax.Precision.DEFAULT)
    out_ref[0, 0] = _pack_edge_pairs(y[:, :HH])
    out_ref[0, 1] = _pack_edge_pairs(y[:, HH:])


def _efp(ef, we):
    B, E, F = ef.shape
    return pl.pallas_call(
        _efp_body,
        grid=(B, E // _BE),
        in_specs=[
            pl.BlockSpec((1, _BE, F), lambda b, i: (b, i, 0)),
            pl.BlockSpec((F, H), lambda b, i: (0, 0)),
        ],
        out_specs=pl.BlockSpec((1, 2, _BE // 2, HH), lambda b, i: (b, 0, i, 0)),
        out_shape=jax.ShapeDtypeStruct((B, 2, E // 2, HH), jnp.int32),
    )(ef, we)


def _post_body(conv_ref, c_ref, s2s_ref, x_ref, w_ref, b_ref, out_ref):
    b = pl.program_id(0)
    cs = c_ref[b, 0, 0]
    conv = jnp.concatenate([conv_ref[0, 0], conv_ref[0, 1]], axis=-1) * cs
    h = jnp.dot(conv, w_ref[0], preferred_element_type=jnp.float32, precision=lax.Precision.DEFAULT) + b_ref[0]
    h = jnp.dot(h, w_ref[1], preferred_element_type=jnp.float32, precision=lax.Precision.DEFAULT) + b_ref[1]
    conv = conv + h
    h = jnp.dot(conv, w_ref[2], preferred_element_type=jnp.float32, precision=lax.Precision.DEFAULT) + b_ref[2]
    h = jnp.dot(h, w_ref[3], preferred_element_type=jnp.float32, precision=lax.Precision.DEFAULT) + b_ref[3]
    conv = conv + h
    out = s2s_ref[0] * conv
    h = jnp.dot(out, w_ref[4], preferred_element_type=jnp.float32, precision=lax.Precision.DEFAULT) + b_ref[4]
    h = jnp.dot(h, w_ref[5], preferred_element_type=jnp.float32, precision=lax.Precision.DEFAULT) + b_ref[5]
    out = out + h
    out_ref[0] = out + x_ref[0]


def _post(conv, c, s2s, scalar, w6, b6):
    B, N, _ = scalar.shape
    return pl.pallas_call(
        _post_body,
        grid=(B, N // _BN),
        in_specs=[
            pl.BlockSpec((1, 2, _BN, HH), lambda b, i: (b, 0, i, 0)),
            pl.BlockSpec(memory_space=pltpu.SMEM),
            pl.BlockSpec((1, _BN, H), lambda b, i: (b, i, 0)),
            pl.BlockSpec((1, _BN, H), lambda b, i: (b, i, 0)),
            pl.BlockSpec((6, H, H), lambda b, i: (0, 0, 0)),
            pl.BlockSpec((6, 1, H), lambda b, i: (0, 0, 0)),
        ],
        out_specs=pl.BlockSpec((1, _BN, H), lambda b, i: (b, i, 0)),
        out_shape=jax.ShapeDtypeStruct((B, N, H), jnp.float32),
    )(conv, c, s2s, scalar, w6, b6)


def _pool_body(x_ref, w_ref, b_ref, out_ref, acc_ref):
    i = pl.program_id(1)
    nb = pl.num_programs(1)
    ps = jnp.sum(x_ref[0], axis=0, keepdims=True)

    @pl.when(i == 0)
    def _():
        acc_ref[...] = ps

    @pl.when(i > 0)
    def _():
        acc_ref[...] = acc_ref[...] + ps

    @pl.when(i == nb - 1)
    def _():
        g = acc_ref[...]
        for j in range(3):
            h = jnp.dot(g, w_ref[2 * j], preferred_element_type=jnp.float32, precision=lax.Precision.DEFAULT) + b_ref[2 * j]
            h = jnp.dot(h, w_ref[2 * j + 1], preferred_element_type=jnp.float32, precision=lax.Precision.DEFAULT) + b_ref[2 * j + 1]
            g = g + h
        out_ref[0] = g


def _pool(scalar, w6, b6):
    B, N, _ = scalar.shape
    return pl.pallas_call(
        _pool_body,
        grid=(B, N // _BN),
        in_specs=[
            pl.BlockSpec((1, _BN, H), lambda b, i: (b, i, 0)),
            pl.BlockSpec((6, H, H), lambda b, i: (0, 0, 0)),
            pl.BlockSpec((6, 1, H), lambda b, i: (0, 0, 0)),
        ],
        out_specs=pl.BlockSpec((1, 1, H), lambda b, i: (b, 0, 0)),
        out_shape=jax.ShapeDtypeStruct((B, 1, H), jnp.float32),
        scratch_shapes=[pltpu.VMEM((1, H), jnp.float32)],
    )(scalar, w6, b6)


@functools.lru_cache(maxsize=None)
def _sc_edge(B, N, E):
    ept = E // 16
    nch = ept // _CH
    rpt = _NP // 16
    zch = rpt // _CH
    nsl = _CH // 16
    mesh = plsc.VectorSubcoreMesh(core_axis_name="c", subcore_axis_name="s")

    @functools.partial(
        pl.kernel,
        out_type=jax.ShapeDtypeStruct((B * 2 * _NP, HH), jnp.float32),
        mesh=mesh,
        compiler_params=pltpu.CompilerParams(needs_layout_passes=False),
        scratch_types=[
            pltpu.VMEM((_CH,), jnp.int32),
            pltpu.VMEM((_CH,), jnp.int32),
            pltpu.VMEM((_CH,), jnp.int32),
            pltpu.VMEM((_CH,), jnp.int32),
            pltpu.VMEM((_CH,), jnp.int32),
            pltpu.VMEM((_CH,), jnp.int32),
            pltpu.VMEM((_CH, HH), jnp.float32),
            pltpu.VMEM((_CH, HH), jnp.float32),
            pltpu.VMEM((_CH // 2, HH), jnp.int32),
            pltpu.VMEM((_CH // 2, HH), jnp.int32),
            pltpu.VMEM_SHARED((_NP, HH), jnp.float32),
            pltpu.SemaphoreType.DMA,
            pltpu.SemaphoreType.DMA,
            pltpu.SemaphoreType.DMA,
            pltpu.SemaphoreType.DMA,
            pltpu.SemaphoreType.DMA,
            pltpu.SemaphoreType.DMA,
            pltpu.SemaphoreType.DMA,
            pltpu.SemaphoreType.DMA,
        ],
    )
    def k(dst_hbm, efp_hbm, col_hbm, row_hbm, out_hbm,
          cix0, cix1, rix0, rix1, rsc0, rsc1, g0, g1, e0, e1, acc,
          gs0, gs1, es0, es1, is0, is1, ss0, ss1):
        c = lax.axis_index("c")
        s = lax.axis_index("s")
        cixs, rixs, rscs = (cix0, cix1), (rix0, rix1), (rsc0, rsc1)
        gbufs, ebufs = (g0, g1), (e0, e1)
        gsems, esems, isems = (gs0, gs1), (es0, es1), (is0, is1)
        ssems = (ss0, ss1)

        for b in range(B):
            tbl_base = (2 * b + c) * N
            efp_base = (2 * b + c) * (E // 2) + s * (ept // 2)
            out_base = (2 * b + c) * _NP + s * rpt
            ib = b * E + s * ept

            def fetch_idx(t, u):
                off = pl.ds(ib + t * _CH, _CH)
                pltpu.async_copy(col_hbm.at[off], cixs[u], isems[u])
                pltpu.async_copy(row_hbm.at[off], rixs[u], isems[u])

            def wait_idx(u):
                d = pltpu.make_async_copy(col_hbm.at[pl.ds(ib, _CH)],
                                          cixs[u], isems[u])
                d.wait()
                d.wait()

            def issue(t, u, first=False):
                if not first:
                    pltpu.make_async_copy(gbufs[u], acc.at[rscs[u]],
                                          ssems[u]).wait()
                wait_idx(u)
                for i2 in range(nsl):
                    sl = pl.ds(i2 * 16, 16)
                    cixs[u][sl] = cixs[u][sl] + tbl_base
                pltpu.async_copy(dst_hbm.at[cixs[u]], gbufs[u], gsems[u])
                pltpu.async_copy(
                    efp_hbm.at[pl.ds(efp_base + t * (_CH // 2), _CH // 2)],
                    ebufs[u], esems[u])

            def process(t, u):
                pltpu.make_async_copy(dst_hbm.at[cixs[u]],
                                      gbufs[u], gsems[u]).wait()
                pltpu.make_async_copy(efp_hbm.at[pl.ds(efp_base, _CH // 2)],
                                      ebufs[u], esems[u]).wait()
                for i2 in range(nsl):
                    sl = pl.ds(i2 * 16, 16)
                    rscs[u][sl] = rixs[u][sl]
                t2 = jnp.where(t + 2 < nch, t + 2, 0)
                fetch_idx(t2, u)

                @plsc.parallel_loop(0, _CH // 2, unroll=4)
                def _mul(jp):
                    for k2 in range(HH // 16):
                        sl = pl.ds(k2 * 16, 16)
                        e32 = plsc.bitcast(ebufs[u][jp, sl], jnp.bfloat16)
                        ea, eb = plsc.unpack(
                            e32, format=plsc.PackFormat.INTERLEAVED)
                        gbufs[u][2 * jp, sl] = gbufs[u][2 * jp, sl] * ea
                        gbufs[u][2 * jp + 1, sl] = (
                            gbufs[u][2 * jp + 1, sl] * eb)

                pltpu.async_copy(gbufs[u], acc.at[rscs[u]], ssems[u],
                                 add=True)

            fetch_idx(0, 0)
            fetch_idx(1, 1)

            @pl.loop(0, _CH)
            def _zrow(i):
                for k2 in range(HH // 16):
                    g0[i, pl.ds(k2 * 16, 16)] = jnp.zeros((16,), jnp.float32)

            for j in range(zch):
                pltpu.sync_copy(g0, acc.at[pl.ds(s * rpt + j * _CH, _CH)])
            plsc.subcore_barrier()

            issue(0, 0, first=True)
            issue(1, 1, first=True)

            @pl.loop(0, (nch - 1) // 2)
            def _pair(i):
                t0 = i * 2
                process(t0, 0)
                issue(t0 + 2, 0)
                process(t0 + 1, 1)
                issue(jnp.where(t0 + 3 < nch, t0 + 3, 0), 1)

            process(nch - 1, 0)
            wait_idx(0)
            pltpu.make_async_copy(g0, acc.at[rsc0], ss0).wait()
            pltpu.make_async_copy(dst_hbm.at[cix1], g1, gs1).wait()
            pltpu.make_async_copy(efp_hbm.at[pl.ds(efp_base, _CH // 2)],
                                  e1, es1).wait()

            plsc.subcore_barrier()
            pltpu.sync_copy(acc.at[pl.ds(s * rpt, rpt)],
                            out_hbm.at[pl.ds(out_base, rpt)])
            plsc.subcore_barrier()

    return k


def _edge_stage(dst, efp, col, row, B, N, E):
    conv = _sc_edge(B, N, E)(dst.reshape(B * 2 * N, HH),
                             efp.reshape(B * 2 * (E // 2), HH), col, row)
    return conv.reshape(B, 2, _NP, HH)


def kernel(scalar, ef, edge_index, C, batch_index, Wd, bd, We):
    B, N, _ = scalar.shape
    E = ef.shape[1]
    layers = We.shape[0]
    ei = edge_index.astype(jnp.int32)
    c32 = C.astype(jnp.float32)
    bd3 = bd.reshape(bd.shape[0], 1, H)
    outs = []
    for b in range(B):
        sb = scalar[b:b + 1]
        efb = ef[b:b + 1]
        rowb = ei[b, 0]
        colb = ei[b, 1]
        cb = c32[b:b + 1]
        for l in range(layers):
            base = l * 10
            s2s, dst = _pre(sb, Wd[base:base + 4], bd3[base:base + 4])
            efp = _efp(efb, We[l])
            conv = _edge_stage(dst, efp, colb, rowb, 1, N, E)
            sb = _post(conv, cb, s2s, sb,
                       Wd[base + 4:base + 10], bd3[base + 4:base + 10])
        pb = layers * 10
        outs.append(_pool(sb, Wd[pb:pb + 6], bd3[pb:pb + 6]))
    return jnp.concatenate(outs, axis=0)


# speedup vs baseline: 1.0047x; 1.0047x over previous
import functools

import jax
import jax.numpy as jnp
from jax import lax
from jax.experimental import pallas as pl
from jax.experimental.pallas import tpu as pltpu
from jax.experimental.pallas import tpu_sc as plsc

H = 256
HH = 128
_BN = 1000
_BE = 2000
_CH = 80
_NP = 10240


def _pack_edge_pairs(x):
    xb = jax.lax.bitcast_convert_type(x, jnp.uint32)
    r = (xb + jnp.uint32(0x7FFF) + ((xb >> 16) & jnp.uint32(1))) >> 16
    r3 = r.reshape(r.shape[0] // 2, 2, r.shape[1])
    c = r3[:, 0, :] | (r3[:, 1, :] << 16)
    return jax.lax.bitcast_convert_type(c, jnp.int32)


def _pre_body(x_ref, w_ref, b_ref, s2s_ref, dst_ref):
    x = x_ref[0]
    h = jnp.dot(x, w_ref[0], preferred_element_type=jnp.float32, precision=lax.Precision.DEFAULT) + b_ref[0]
    s2s = jnp.dot(h, w_ref[1], preferred_element_type=jnp.float32, precision=lax.Precision.DEFAULT) + b_ref[1]
    s2s_ref[0] = s2s
    h = jnp.dot(x, w_ref[2], preferred_element_type=jnp.float32, precision=lax.Precision.DEFAULT) + b_ref[2]
    dst = jnp.dot(h, w_ref[3], preferred_element_type=jnp.float32, precision=lax.Precision.DEFAULT) + b_ref[3]
    dst_ref[0, 0] = dst[:, :HH]
    dst_ref[0, 1] = dst[:, HH:]


def _pre(scalar, w4, b4):
    B, N, _ = scalar.shape
    return pl.pallas_call(
        _pre_body,
        grid=(B, N // _BN),
        in_specs=[
            pl.BlockSpec((1, _BN, H), lambda b, i: (b, i, 0)),
            pl.BlockSpec((4, H, H), lambda b, i: (0, 0, 0)),
            pl.BlockSpec((4, 1, H), lambda b, i: (0, 0, 0)),
        ],
        out_specs=[
            pl.BlockSpec((1, _BN, H), lambda b, i: (b, i, 0)),
            pl.BlockSpec((1, 2, _BN, HH), lambda b, i: (b, 0, i, 0)),
        ],
        out_shape=[
            jax.ShapeDtypeStruct((B, N, H), jnp.float32),
            jax.ShapeDtypeStruct((B, 2, N, HH), jnp.float32),
        ],
    )(scalar, w4, b4)


def _efp_body(ef_ref, we_ref, out_ref):
    y = jnp.dot(ef_ref[0], we_ref[...], preferred_element_type=jnp.float32, precision=lax.Precision.DEFAULT)
    out_ref[0, 0] = _pack_edge_pairs(y[:, :HH])
    out_ref[0, 1] = _pack_edge_pairs(y[:, HH:])


def _efp(ef, we):
    B, E, F = ef.shape
    return pl.pallas_call(
        _efp_body,
        grid=(B, E // _BE),
        in_specs=[
            pl.BlockSpec((1, _BE, F), lambda b, i: (b, i, 0)),
            pl.BlockSpec((F, H), lambda b, i: (0, 0)),
        ],
        out_specs=pl.BlockSpec((1, 2, _BE // 2, HH), lambda b, i: (b, 0, i, 0)),
        out_shape=jax.ShapeDtypeStruct((B, 2, E // 2, HH), jnp.int32),
    )(ef, we)


def _post_body(conv_ref, c_ref, s2s_ref, x_ref, w_ref, b_ref, out_ref):
    b = pl.program_id(0)
    cs = c_ref[b, 0, 0]
    conv = jnp.concatenate([conv_ref[0, 0], conv_ref[0, 1]], axis=-1) * cs
    h = jnp.dot(conv, w_ref[0], preferred_element_type=jnp.float32, precision=lax.Precision.DEFAULT) + b_ref[0]
    h = jnp.dot(h, w_ref[1], preferred_element_type=jnp.float32, precision=lax.Precision.DEFAULT) + b_ref[1]
    conv = conv + h
    h = jnp.dot(conv, w_ref[2], preferred_element_type=jnp.float32, precision=lax.Precision.DEFAULT) + b_ref[2]
    h = jnp.dot(h, w_ref[3], preferred_element_type=jnp.float32, precision=lax.Precision.DEFAULT) + b_ref[3]
    conv = conv + h
    out = s2s_ref[0] * conv
    h = jnp.dot(out, w_ref[4], preferred_element_type=jnp.float32, precision=lax.Precision.DEFAULT) + b_ref[4]
    h = jnp.dot(h, w_ref[5], preferred_element_type=jnp.float32, precision=lax.Precision.DEFAULT) + b_ref[5]
    out = out + h
    out_ref[0] = out + x_ref[0]


def _post(conv, c, s2s, scalar, w6, b6):
    B, N, _ = scalar.shape
    return pl.pallas_call(
        _post_body,
        grid=(B, N // _BN),
        in_specs=[
            pl.BlockSpec((1, 2, _BN, HH), lambda b, i: (b, 0, i, 0)),
            pl.BlockSpec(memory_space=pltpu.SMEM),
            pl.BlockSpec((1, _BN, H), lambda b, i: (b, i, 0)),
            pl.BlockSpec((1, _BN, H), lambda b, i: (b, i, 0)),
            pl.BlockSpec((6, H, H), lambda b, i: (0, 0, 0)),
            pl.BlockSpec((6, 1, H), lambda b, i: (0, 0, 0)),
        ],
        out_specs=pl.BlockSpec((1, _BN, H), lambda b, i: (b, i, 0)),
        out_shape=jax.ShapeDtypeStruct((B, N, H), jnp.float32),
    )(conv, c, s2s, scalar, w6, b6)


def _pool_body(x_ref, w_ref, b_ref, out_ref, acc_ref):
    i = pl.program_id(1)
    nb = pl.num_programs(1)
    ps = jnp.sum(x_ref[0], axis=0, keepdims=True)

    @pl.when(i == 0)
    def _():
        acc_ref[...] = ps

    @pl.when(i > 0)
    def _():
        acc_ref[...] = acc_ref[...] + ps

    @pl.when(i == nb - 1)
    def _():
        g = acc_ref[...]
        for j in range(3):
            h = jnp.dot(g, w_ref[2 * j], preferred_element_type=jnp.float32, precision=lax.Precision.DEFAULT) + b_ref[2 * j]
            h = jnp.dot(h, w_ref[2 * j + 1], preferred_element_type=jnp.float32, precision=lax.Precision.DEFAULT) + b_ref[2 * j + 1]
            g = g + h
        out_ref[0] = g


def _pool(scalar, w6, b6):
    B, N, _ = scalar.shape
    return pl.pallas_call(
        _pool_body,
        grid=(B, N // _BN),
        in_specs=[
            pl.BlockSpec((1, _BN, H), lambda b, i: (b, i, 0)),
            pl.BlockSpec((6, H, H), lambda b, i: (0, 0, 0)),
            pl.BlockSpec((6, 1, H), lambda b, i: (0, 0, 0)),
        ],
        out_specs=pl.BlockSpec((1, 1, H), lambda b, i: (b, 0, 0)),
        out_shape=jax.ShapeDtypeStruct((B, 1, H), jnp.float32),
        scratch_shapes=[pltpu.VMEM((1, H), jnp.float32)],
    )(scalar, w6, b6)


@functools.lru_cache(maxsize=None)
def _sc_edge(B, N, E):
    ept = E // 16
    nch = ept // _CH
    rpt = _NP // 16
    zch = rpt // _CH
    nsl = _CH // 16
    mesh = plsc.VectorSubcoreMesh(core_axis_name="c", subcore_axis_name="s")

    @functools.partial(
        pl.kernel,
        out_type=jax.ShapeDtypeStruct((B * 2 * _NP, HH), jnp.float32),
        mesh=mesh,
        compiler_params=pltpu.CompilerParams(needs_layout_passes=False),
        scratch_types=[
            pltpu.VMEM((_CH,), jnp.int32),
            pltpu.VMEM((_CH,), jnp.int32),
            pltpu.VMEM((_CH,), jnp.int32),
            pltpu.VMEM((_CH,), jnp.int32),
            pltpu.VMEM((_CH,), jnp.int32),
            pltpu.VMEM((_CH,), jnp.int32),
            pltpu.VMEM((_CH, HH), jnp.float32),
            pltpu.VMEM((_CH, HH), jnp.float32),
            pltpu.VMEM((_CH // 2, HH), jnp.int32),
            pltpu.VMEM((_CH // 2, HH), jnp.int32),
            pltpu.VMEM_SHARED((_NP, HH), jnp.float32),
            pltpu.SemaphoreType.DMA,
            pltpu.SemaphoreType.DMA,
            pltpu.SemaphoreType.DMA,
            pltpu.SemaphoreType.DMA,
            pltpu.SemaphoreType.DMA,
            pltpu.SemaphoreType.DMA,
            pltpu.SemaphoreType.DMA,
            pltpu.SemaphoreType.DMA,
        ],
    )
    def k(dst_hbm, efp_hbm, col_hbm, row_hbm, out_hbm,
          cix0, cix1, rix0, rix1, rsc0, rsc1, g0, g1, e0, e1, acc,
          gs0, gs1, es0, es1, is0, is1, ss0, ss1):
        c = lax.axis_index("c")
        s = lax.axis_index("s")
        cixs, rixs, rscs = (cix0, cix1), (rix0, rix1), (rsc0, rsc1)
        gbufs, ebufs = (g0, g1), (e0, e1)
        gsems, esems, isems = (gs0, gs1), (es0, es1), (is0, is1)
        ssems = (ss0, ss1)

        for b in range(B):
            tbl_base = (2 * b + c) * N
            efp_base = (2 * b + c) * (E // 2) + s * (ept // 2)
            out_base = (2 * b + c) * _NP + s * rpt
            ib = b * E + s * ept

            def fetch_idx(t, u):
                off = pl.ds(ib + t * _CH, _CH)
                pltpu.async_copy(col_hbm.at[off], cixs[u], isems[u])
                pltpu.async_copy(row_hbm.at[off], rixs[u], isems[u])

            def wait_idx(u):
                d = pltpu.make_async_copy(col_hbm.at[pl.ds(ib, _CH)],
                                          cixs[u], isems[u])
                d.wait()
                d.wait()

            def issue(t, u, first=False):
                if not first:
                    pltpu.make_async_copy(gbufs[u], acc.at[rscs[u]],
                                          ssems[u]).wait()
                wait_idx(u)
                for i2 in range(nsl):
                    sl = pl.ds(i2 * 16, 16)
                    cixs[u][sl] = cixs[u][sl] + tbl_base
                pltpu.async_copy(dst_hbm.at[cixs[u]], gbufs[u], gsems[u])
                pltpu.async_copy(
                    efp_hbm.at[pl.ds(efp_base + t * (_CH // 2), _CH // 2)],
                    ebufs[u], esems[u])

            def process(t, u):
                pltpu.make_async_copy(dst_hbm.at[cixs[u]],
                                      gbufs[u], gsems[u]).wait()
                pltpu.make_async_copy(efp_hbm.at[pl.ds(efp_base, _CH // 2)],
                                      ebufs[u], esems[u]).wait()
                for i2 in range(nsl):
                    sl = pl.ds(i2 * 16, 16)
                    rscs[u][sl] = rixs[u][sl]
                t2 = jnp.where(t + 2 < nch, t + 2, 0)
                fetch_idx(t2, u)

                @plsc.parallel_loop(0, _CH // 2, unroll=2)
                def _mul(jp):
                    for k2 in range(HH // 16):
                        sl = pl.ds(k2 * 16, 16)
                        e32 = plsc.bitcast(ebufs[u][jp, sl], jnp.bfloat16)
                        ea, eb = plsc.unpack(
                            e32, format=plsc.PackFormat.INTERLEAVED)
                        gbufs[u][2 * jp, sl] = gbufs[u][2 * jp, sl] * ea
                        gbufs[u][2 * jp + 1, sl] = (
                            gbufs[u][2 * jp + 1, sl] * eb)

                pltpu.async_copy(gbufs[u], acc.at[rscs[u]], ssems[u],
                                 add=True)

            fetch_idx(0, 0)
            fetch_idx(1, 1)

            @pl.loop(0, _CH)
            def _zrow(i):
                for k2 in range(HH // 16):
                    g0[i, pl.ds(k2 * 16, 16)] = jnp.zeros((16,), jnp.float32)

            for j in range(zch):
                pltpu.sync_copy(g0, acc.at[pl.ds(s * rpt + j * _CH, _CH)])
            plsc.subcore_barrier()

            issue(0, 0, first=True)
            issue(1, 1, first=True)

            @pl.loop(0, (nch - 1) // 2)
            def _pair(i):
                t0 = i * 2
                process(t0, 0)
                issue(t0 + 2, 0)
                process(t0 + 1, 1)
                issue(jnp.where(t0 + 3 < nch, t0 + 3, 0), 1)

            process(nch - 1, 0)
            wait_idx(0)
            pltpu.make_async_copy(g0, acc.at[rsc0], ss0).wait()
            pltpu.make_async_copy(dst_hbm.at[cix1], g1, gs1).wait()
            pltpu.make_async_copy(efp_hbm.at[pl.ds(efp_base, _CH // 2)],
                                  e1, es1).wait()

            plsc.subcore_barrier()
            pltpu.sync_copy(acc.at[pl.ds(s * rpt, rpt)],
                            out_hbm.at[pl.ds(out_base, rpt)])
            plsc.subcore_barrier()

    return k


def _edge_stage(dst, efp, col, row, B, N, E):
    conv = _sc_edge(B, N, E)(dst.reshape(B * 2 * N, HH),
                             efp.reshape(B * 2 * (E // 2), HH), col, row)
    return conv.reshape(B, 2, _NP, HH)


def kernel(scalar, ef, edge_index, C, batch_index, Wd, bd, We):
    B, N, _ = scalar.shape
    E = ef.shape[1]
    layers = We.shape[0]
    ei = edge_index.astype(jnp.int32)
    c32 = C.astype(jnp.float32)
    bd3 = bd.reshape(bd.shape[0], 1, H)
    sb = [scalar[b:b + 1] for b in range(B)]
    efb = [ef[b:b + 1] for b in range(B)]
    rowb = [ei[b, 0] for b in range(B)]
    colb = [ei[b, 1] for b in range(B)]
    cb = [c32[b:b + 1] for b in range(B)]
    s2s = [None] * B
    conv = [None] * B
    for l in range(layers):
        base = l * 10
        for b in range(B):
            s2s[b], dst = _pre(sb[b], Wd[base:base + 4], bd3[base:base + 4])
            efp = _efp(efb[b], We[l])
            conv[b] = _edge_stage(dst, efp, colb[b], rowb[b], 1, N, E)
        for b in range(B):
            sb[b] = _post(conv[b], cb[b], s2s[b], sb[b],
                          Wd[base + 4:base + 10], bd3[base + 4:base + 10])
    pb = layers * 10
    outs = [_pool(sb[b], Wd[pb:pb + 6], bd3[pb:pb + 6]) for b in range(B)]
    return jnp.concatenate(outs, axis=0)


# speedup vs baseline: 1.0405x; 1.0356x over previous
import functools

import jax
import jax.numpy as jnp
from jax import lax
from jax.experimental import pallas as pl
from jax.experimental.pallas import tpu as pltpu
from jax.experimental.pallas import tpu_sc as plsc

H = 256
HH = 128
_BN = 1000
_BE = 2000
_CH = 80
_NP = 10240


def _pack_edge_pairs(x):
    xb = jax.lax.bitcast_convert_type(x, jnp.uint32)
    r = (xb + jnp.uint32(0x7FFF) + ((xb >> 16) & jnp.uint32(1))) >> 16
    r3 = r.reshape(r.shape[0] // 2, 2, r.shape[1])
    c = r3[:, 0, :] | (r3[:, 1, :] << 16)
    return jax.lax.bitcast_convert_type(c, jnp.int32)


def _pre_body(x_ref, w_ref, b_ref, s2s_ref, dst_ref):
    x = x_ref[0]
    h = jnp.dot(x, w_ref[0], preferred_element_type=jnp.float32, precision=lax.Precision.DEFAULT) + b_ref[0]
    s2s = jnp.dot(h, w_ref[1], preferred_element_type=jnp.float32, precision=lax.Precision.DEFAULT) + b_ref[1]
    s2s_ref[0] = s2s
    h = jnp.dot(x, w_ref[2], preferred_element_type=jnp.float32, precision=lax.Precision.DEFAULT) + b_ref[2]
    dst = jnp.dot(h, w_ref[3], preferred_element_type=jnp.float32, precision=lax.Precision.DEFAULT) + b_ref[3]
    dst_ref[0, 0] = dst[:, :HH]
    dst_ref[0, 1] = dst[:, HH:]


def _pre(scalar, w4, b4):
    B, N, _ = scalar.shape
    return pl.pallas_call(
        _pre_body,
        grid=(B, N // _BN),
        in_specs=[
            pl.BlockSpec((1, _BN, H), lambda b, i: (b, i, 0)),
            pl.BlockSpec((4, H, H), lambda b, i: (0, 0, 0)),
            pl.BlockSpec((4, 1, H), lambda b, i: (0, 0, 0)),
        ],
        out_specs=[
            pl.BlockSpec((1, _BN, H), lambda b, i: (b, i, 0)),
            pl.BlockSpec((1, 2, _BN, HH), lambda b, i: (b, 0, i, 0)),
        ],
        out_shape=[
            jax.ShapeDtypeStruct((B, N, H), jnp.float32),
            jax.ShapeDtypeStruct((B, 2, N, HH), jnp.float32),
        ],
    )(scalar, w4, b4)


def _efp_body(ef_ref, we_ref, out_ref):
    y = jnp.dot(ef_ref[0], we_ref[...], preferred_element_type=jnp.float32, precision=lax.Precision.DEFAULT)
    out_ref[0, 0] = _pack_edge_pairs(y[:, :HH])
    out_ref[0, 1] = _pack_edge_pairs(y[:, HH:])


def _efp(ef, we):
    B, E, F = ef.shape
    return pl.pallas_call(
        _efp_body,
        grid=(B, E // _BE),
        in_specs=[
            pl.BlockSpec((1, _BE, F), lambda b, i: (b, i, 0)),
            pl.BlockSpec((F, H), lambda b, i: (0, 0)),
        ],
        out_specs=pl.BlockSpec((1, 2, _BE // 2, HH), lambda b, i: (b, 0, i, 0)),
        out_shape=jax.ShapeDtypeStruct((B, 2, E // 2, HH), jnp.int32),
    )(ef, we)


def _post_body(conv_ref, c_ref, s2s_ref, x_ref, w_ref, b_ref, out_ref):
    b = pl.program_id(0)
    cs = c_ref[b, 0, 0]
    conv = jnp.concatenate([conv_ref[0, 0], conv_ref[0, 1]], axis=-1) * cs
    h = jnp.dot(conv, w_ref[0], preferred_element_type=jnp.float32, precision=lax.Precision.DEFAULT) + b_ref[0]
    h = jnp.dot(h, w_ref[1], preferred_element_type=jnp.float32, precision=lax.Precision.DEFAULT) + b_ref[1]
    conv = conv + h
    h = jnp.dot(conv, w_ref[2], preferred_element_type=jnp.float32, precision=lax.Precision.DEFAULT) + b_ref[2]
    h = jnp.dot(h, w_ref[3], preferred_element_type=jnp.float32, precision=lax.Precision.DEFAULT) + b_ref[3]
    conv = conv + h
    out = s2s_ref[0] * conv
    h = jnp.dot(out, w_ref[4], preferred_element_type=jnp.float32, precision=lax.Precision.DEFAULT) + b_ref[4]
    h = jnp.dot(h, w_ref[5], preferred_element_type=jnp.float32, precision=lax.Precision.DEFAULT) + b_ref[5]
    out = out + h
    out_ref[0] = out + x_ref[0]


def _post(conv, c, s2s, scalar, w6, b6):
    B, N, _ = scalar.shape
    return pl.pallas_call(
        _post_body,
        grid=(B, N // _BN),
        in_specs=[
            pl.BlockSpec((1, 2, _BN, HH), lambda b, i: (b, 0, i, 0)),
            pl.BlockSpec(memory_space=pltpu.SMEM),
            pl.BlockSpec((1, _BN, H), lambda b, i: (b, i, 0)),
            pl.BlockSpec((1, _BN, H), lambda b, i: (b, i, 0)),
            pl.BlockSpec((6, H, H), lambda b, i: (0, 0, 0)),
            pl.BlockSpec((6, 1, H), lambda b, i: (0, 0, 0)),
        ],
        out_specs=pl.BlockSpec((1, _BN, H), lambda b, i: (b, i, 0)),
        out_shape=jax.ShapeDtypeStruct((B, N, H), jnp.float32),
    )(conv, c, s2s, scalar, w6, b6)


def _fused_body(conv_ref, c_ref, s2s_ref, x_ref, w_ref, b_ref,
                sc_ref, s2s2_ref, dst_ref):
    cs = c_ref[0, 0, 0]
    conv = jnp.concatenate([conv_ref[0, 0], conv_ref[0, 1]], axis=-1) * cs
    h = jnp.dot(conv, w_ref[0], preferred_element_type=jnp.float32, precision=lax.Precision.DEFAULT) + b_ref[0]
    h = jnp.dot(h, w_ref[1], preferred_element_type=jnp.float32, precision=lax.Precision.DEFAULT) + b_ref[1]
    conv = conv + h
    h = jnp.dot(conv, w_ref[2], preferred_element_type=jnp.float32, precision=lax.Precision.DEFAULT) + b_ref[2]
    h = jnp.dot(h, w_ref[3], preferred_element_type=jnp.float32, precision=lax.Precision.DEFAULT) + b_ref[3]
    conv = conv + h
    out = s2s_ref[0] * conv
    h = jnp.dot(out, w_ref[4], preferred_element_type=jnp.float32, precision=lax.Precision.DEFAULT) + b_ref[4]
    h = jnp.dot(h, w_ref[5], preferred_element_type=jnp.float32, precision=lax.Precision.DEFAULT) + b_ref[5]
    x = out + h + x_ref[0]
    sc_ref[0] = x
    h = jnp.dot(x, w_ref[6], preferred_element_type=jnp.float32, precision=lax.Precision.DEFAULT) + b_ref[6]
    s2s2_ref[0] = jnp.dot(h, w_ref[7], preferred_element_type=jnp.float32, precision=lax.Precision.DEFAULT) + b_ref[7]
    h = jnp.dot(x, w_ref[8], preferred_element_type=jnp.float32, precision=lax.Precision.DEFAULT) + b_ref[8]
    dst = jnp.dot(h, w_ref[9], preferred_element_type=jnp.float32, precision=lax.Precision.DEFAULT) + b_ref[9]
    dst_ref[0, 0] = dst[:, :HH]
    dst_ref[0, 1] = dst[:, HH:]


def _fused(conv, c, s2s, scalar, w10, b10):
    B, N, _ = scalar.shape
    return pl.pallas_call(
        _fused_body,
        grid=(B, N // _BN),
        in_specs=[
            pl.BlockSpec((1, 2, _BN, HH), lambda b, i: (b, 0, i, 0)),
            pl.BlockSpec(memory_space=pltpu.SMEM),
            pl.BlockSpec((1, _BN, H), lambda b, i: (b, i, 0)),
            pl.BlockSpec((1, _BN, H), lambda b, i: (b, i, 0)),
            pl.BlockSpec((10, H, H), lambda b, i: (0, 0, 0)),
            pl.BlockSpec((10, 1, H), lambda b, i: (0, 0, 0)),
        ],
        out_specs=[
            pl.BlockSpec((1, _BN, H), lambda b, i: (b, i, 0)),
            pl.BlockSpec((1, _BN, H), lambda b, i: (b, i, 0)),
            pl.BlockSpec((1, 2, _BN, HH), lambda b, i: (b, 0, i, 0)),
        ],
        out_shape=[
            jax.ShapeDtypeStruct((B, N, H), jnp.float32),
            jax.ShapeDtypeStruct((B, N, H), jnp.float32),
            jax.ShapeDtypeStruct((B, 2, N, HH), jnp.float32),
        ],
    )(conv, c, s2s, scalar, w10, b10)


def _pool_body(x_ref, w_ref, b_ref, out_ref, acc_ref):
    i = pl.program_id(1)
    nb = pl.num_programs(1)
    ps = jnp.sum(x_ref[0], axis=0, keepdims=True)

    @pl.when(i == 0)
    def _():
        acc_ref[...] = ps

    @pl.when(i > 0)
    def _():
        acc_ref[...] = acc_ref[...] + ps

    @pl.when(i == nb - 1)
    def _():
        g = acc_ref[...]
        for j in range(3):
            h = jnp.dot(g, w_ref[2 * j], preferred_element_type=jnp.float32, precision=lax.Precision.DEFAULT) + b_ref[2 * j]
            h = jnp.dot(h, w_ref[2 * j + 1], preferred_element_type=jnp.float32, precision=lax.Precision.DEFAULT) + b_ref[2 * j + 1]
            g = g + h
        out_ref[0] = g


def _pool(scalar, w6, b6):
    B, N, _ = scalar.shape
    return pl.pallas_call(
        _pool_body,
        grid=(B, N // _BN),
        in_specs=[
            pl.BlockSpec((1, _BN, H), lambda b, i: (b, i, 0)),
            pl.BlockSpec((6, H, H), lambda b, i: (0, 0, 0)),
            pl.BlockSpec((6, 1, H), lambda b, i: (0, 0, 0)),
        ],
        out_specs=pl.BlockSpec((1, 1, H), lambda b, i: (b, 0, 0)),
        out_shape=jax.ShapeDtypeStruct((B, 1, H), jnp.float32),
        scratch_shapes=[pltpu.VMEM((1, H), jnp.float32)],
    )(scalar, w6, b6)


@functools.lru_cache(maxsize=None)
def _sc_edge(B, N, E):
    ept = E // 16
    nch = ept // _CH
    rpt = _NP // 16
    zch = rpt // _CH
    nsl = _CH // 16
    mesh = plsc.VectorSubcoreMesh(core_axis_name="c", subcore_axis_name="s")

    @functools.partial(
        pl.kernel,
        out_type=jax.ShapeDtypeStruct((B * 2 * _NP, HH), jnp.float32),
        mesh=mesh,
        compiler_params=pltpu.CompilerParams(needs_layout_passes=False),
        scratch_types=[
            pltpu.VMEM((_CH,), jnp.int32),
            pltpu.VMEM((_CH,), jnp.int32),
            pltpu.VMEM((_CH,), jnp.int32),
            pltpu.VMEM((_CH,), jnp.int32),
            pltpu.VMEM((_CH,), jnp.int32),
            pltpu.VMEM((_CH,), jnp.int32),
            pltpu.VMEM((_CH, HH), jnp.float32),
            pltpu.VMEM((_CH, HH), jnp.float32),
            pltpu.VMEM((_CH // 2, HH), jnp.int32),
            pltpu.VMEM((_CH // 2, HH), jnp.int32),
            pltpu.VMEM_SHARED((_NP, HH), jnp.float32),
            pltpu.SemaphoreType.DMA,
            pltpu.SemaphoreType.DMA,
            pltpu.SemaphoreType.DMA,
            pltpu.SemaphoreType.DMA,
            pltpu.SemaphoreType.DMA,
            pltpu.SemaphoreType.DMA,
            pltpu.SemaphoreType.DMA,
            pltpu.SemaphoreType.DMA,
        ],
    )
    def k(dst_hbm, efp_hbm, col_hbm, row_hbm, out_hbm,
          cix0, cix1, rix0, rix1, rsc0, rsc1, g0, g1, e0, e1, acc,
          gs0, gs1, es0, es1, is0, is1, ss0, ss1):
        c = lax.axis_index("c")
        s = lax.axis_index("s")
        cixs, rixs, rscs = (cix0, cix1), (rix0, rix1), (rsc0, rsc1)
        gbufs, ebufs = (g0, g1), (e0, e1)
        gsems, esems, isems = (gs0, gs1), (es0, es1), (is0, is1)
        ssems = (ss0, ss1)

        for b in range(B):
            tbl_base = (2 * b + c) * N
            efp_base = (2 * b + c) * (E // 2) + s * (ept // 2)
            out_base = (2 * b + c) * _NP + s * rpt
            ib = b * E + s * ept

            def fetch_idx(t, u):
                off = pl.ds(ib + t * _CH, _CH)
                pltpu.async_copy(col_hbm.at[off], cixs[u], isems[u])
                pltpu.async_copy(row_hbm.at[off], rixs[u], isems[u])

            def wait_idx(u):
                d = pltpu.make_async_copy(col_hbm.at[pl.ds(ib, _CH)],
                                          cixs[u], isems[u])
                d.wait()
                d.wait()

            def issue(t, u, first=False):
                if not first:
                    pltpu.make_async_copy(gbufs[u], acc.at[rscs[u]],
                                          ssems[u]).wait()
                wait_idx(u)
                for i2 in range(nsl):
                    sl = pl.ds(i2 * 16, 16)
                    cixs[u][sl] = cixs[u][sl] + tbl_base
                pltpu.async_copy(dst_hbm.at[cixs[u]], gbufs[u], gsems[u])
                pltpu.async_copy(
                    efp_hbm.at[pl.ds(efp_base + t * (_CH // 2), _CH // 2)],
                    ebufs[u], esems[u])

            def process(t, u):
                pltpu.make_async_copy(dst_hbm.at[cixs[u]],
                                      gbufs[u], gsems[u]).wait()
                pltpu.make_async_copy(efp_hbm.at[pl.ds(efp_base, _CH // 2)],
                                      ebufs[u], esems[u]).wait()
                for i2 in range(nsl):
                    sl = pl.ds(i2 * 16, 16)
                    rscs[u][sl] = rixs[u][sl]
                t2 = jnp.where(t + 2 < nch, t + 2, 0)
                fetch_idx(t2, u)

                @plsc.parallel_loop(0, _CH // 2, unroll=2)
                def _mul(jp):
                    for k2 in range(HH // 16):
                        sl = pl.ds(k2 * 16, 16)
                        e32 = plsc.bitcast(ebufs[u][jp, sl], jnp.bfloat16)
                        ea, eb = plsc.unpack(
                            e32, format=plsc.PackFormat.INTERLEAVED)
                        gbufs[u][2 * jp, sl] = gbufs[u][2 * jp, sl] * ea
                        gbufs[u][2 * jp + 1, sl] = (
                            gbufs[u][2 * jp + 1, sl] * eb)

                pltpu.async_copy(gbufs[u], acc.at[rscs[u]], ssems[u],
                                 add=True)

            fetch_idx(0, 0)
            fetch_idx(1, 1)

            @pl.loop(0, _CH)
            def _zrow(i):
                for k2 in range(HH // 16):
                    g0[i, pl.ds(k2 * 16, 16)] = jnp.zeros((16,), jnp.float32)

            for j in range(zch):
                pltpu.sync_copy(g0, acc.at[pl.ds(s * rpt + j * _CH, _CH)])
            plsc.subcore_barrier()

            issue(0, 0, first=True)
            issue(1, 1, first=True)

            @pl.loop(0, (nch - 1) // 2)
            def _pair(i):
                t0 = i * 2
                process(t0, 0)
                issue(t0 + 2, 0)
                process(t0 + 1, 1)
                issue(jnp.where(t0 + 3 < nch, t0 + 3, 0), 1)

            process(nch - 1, 0)
            wait_idx(0)
            pltpu.make_async_copy(g0, acc.at[rsc0], ss0).wait()
            pltpu.make_async_copy(dst_hbm.at[cix1], g1, gs1).wait()
            pltpu.make_async_copy(efp_hbm.at[pl.ds(efp_base, _CH // 2)],
                                  e1, es1).wait()

            plsc.subcore_barrier()
            pltpu.sync_copy(acc.at[pl.ds(s * rpt, rpt)],
                            out_hbm.at[pl.ds(out_base, rpt)])
            plsc.subcore_barrier()

    return k


def _edge_stage(dst, efp, col, row, B, N, E):
    conv = _sc_edge(B, N, E)(dst.reshape(B * 2 * N, HH),
                             efp.reshape(B * 2 * (E // 2), HH), col, row)
    return conv.reshape(B, 2, _NP, HH)


def kernel(scalar, ef, edge_index, C, batch_index, Wd, bd, We):
    B, N, _ = scalar.shape
    E = ef.shape[1]
    layers = We.shape[0]
    ei = edge_index.astype(jnp.int32)
    c32 = C.astype(jnp.float32)
    bd3 = bd.reshape(bd.shape[0], 1, H)
    sb = [scalar[b:b + 1] for b in range(B)]
    efb = [ef[b:b + 1] for b in range(B)]
    rowb = [ei[b, 0] for b in range(B)]
    colb = [ei[b, 1] for b in range(B)]
    cb = [c32[b:b + 1] for b in range(B)]
    s2s = [None] * B
    dst = [None] * B
    conv = [None] * B
    for b in range(B):
        s2s[b], dst[b] = _pre(sb[b], Wd[0:4], bd3[0:4])
    for l in range(layers):
        base = l * 10
        for b in range(B):
            efp = _efp(efb[b], We[l])
            conv[b] = _edge_stage(dst[b], efp, colb[b], rowb[b], 1, N, E)
        if l < layers - 1:
            for b in range(B):
                sb[b], s2s[b], dst[b] = _fused(
                    conv[b], cb[b], s2s[b], sb[b],
                    Wd[base + 4:base + 14], bd3[base + 4:base + 14])
        else:
            for b in range(B):
                sb[b] = _post(conv[b], cb[b], s2s[b], sb[b],
                              Wd[base + 4:base + 10], bd3[base + 4:base + 10])
    pb = layers * 10
    outs = [_pool(sb[b], Wd[pb:pb + 6], bd3[pb:pb + 6]) for b in range(B)]
    return jnp.concatenate(outs, axis=0)
